# baseline (speedup 1.0000x reference)
import functools

import jax
import jax.numpy as jnp
from jax import lax
from jax.experimental import pallas as pl
from jax.experimental.pallas import tpu as pltpu
from jax.experimental.pallas import tpu_sc as plsc

F32 = jnp.float32


def _mm(a, w):
    return lax.dot_general(a, w, (((1,), (0,)), ((), ())),
                           precision=lax.Precision.HIGHEST,
                           preferred_element_type=F32)


def _rss(vh, nch):
    s2 = vh[:, :nch] ** 2 + vh[:, nch:2 * nch] ** 2 + vh[:, 2 * nch:] ** 2
    return jnp.sqrt(jnp.maximum(s2, 1e-8))


def _ln_s(s, g, b):
    mu = jnp.mean(s, axis=-1, keepdims=True)
    var = jnp.mean((s - mu) ** 2, axis=-1, keepdims=True)
    return (s - mu) / jnp.sqrt(var + 1e-5) * g + b


def _ln_v(v, nch):
    s2 = jnp.maximum(v[:, :nch] ** 2 + v[:, nch:2 * nch] ** 2
                     + v[:, 2 * nch:] ** 2, 1e-8)
    rms = jnp.sqrt(jnp.mean(s2, axis=-1, keepdims=True))
    return v / rms


def _node_math(x, w):
    xs, xv, nt = x[:, 0:6], x[:, 6:15], x[:, 15]
    vh = _mm(xv, w['np_wh'])
    vn = _rss(vh, 8)
    s = _mm(jnp.concatenate([xs, vn], 1), w['np_ws']) + w['np_bs']
    v = _mm(vh, w['np_wv'])
    s = _ln_s(s, w['npln_g'], w['npln_b'])
    v = _ln_v(v, 8)
    ids = lax.broadcasted_iota(jnp.int32, (x.shape[0], 20), 1).astype(F32)
    oh = (nt[:, None] == ids).astype(F32)
    s = jnp.concatenate([_mm(oh, w['ntype_emb']), s], 1)
    s = _ln_s(s, w['gnln_g'], w['gnln_b'])
    v = _ln_v(v, 8)
    vh = _mm(v, w['gn_wh'])
    vn = _rss(vh, 8)
    s = _mm(jnp.concatenate([s, vn], 1), w['gn_ws']) + w['gn_bs']
    v = _mm(vh, w['gn_wv'])
    return jnp.concatenate(
        [s, v, jnp.zeros((x.shape[0], 40), F32)], 1)


def _edge_math(gs, gd, ein, w):
    s_src, v_src = gs[:, 0:64], gs[:, 64:88]
    s_dst, v_dst = gd[:, 0:64], gd[:, 64:88]
    es_a, ev, et = ein[:, 0:32], ein[:, 32:35], ein[:, 35]
    vh = ev * w['ep_wh']
    vn = _rss(vh, 1)
    es = _mm(jnp.concatenate([es_a, vn], 1), w['ep_ws']) + w['ep_bs']
    ev1 = vh * w['ep_wv']
    es = _ln_s(es, w['epln_g'], w['epln_b'])
    ev1 = _ln_v(ev1, 1)
    ids = lax.broadcasted_iota(jnp.int32, (ein.shape[0], 4), 1).astype(F32)
    oh = (et[:, None] == ids).astype(F32)
    es = jnp.concatenate([_mm(oh, w['etype_emb']), es], 1)
    es = _ln_s(es, w['geln_g'], w['geln_b'])
    ev1 = _ln_v(ev1, 1)
    vh = ev1 * w['ge_wh']
    vn = _rss(vh, 1)
    es = _mm(jnp.concatenate([es, vn], 1), w['ge_ws']) + w['ge_bs']
    ev2 = vh * w['ge_wv']
    ms = jnp.concatenate([s_src, es, s_dst], 1)
    mv = jnp.concatenate([
        v_src[:, 0:8], ev2[:, 0:1], v_dst[:, 0:8],
        v_src[:, 8:16], ev2[:, 1:2], v_dst[:, 8:16],
        v_src[:, 16:24], ev2[:, 2:3], v_dst[:, 16:24]], 1)
    vh = _mm(mv, w['m0_wh'])
    vn = _rss(vh, 17)
    ms = _mm(jnp.concatenate([ms, vn], 1), w['m0_ws']) + w['m0_bs']
    v = _mm(vh, w['m0_wv'])
    vh = _mm(v, w['m1_wh'])
    vn = _rss(vh, 8)
    ms = _mm(jnp.concatenate([ms, vn], 1), w['m1_ws']) + w['m1_bs']
    v = _mm(vh, w['m1_wv'])
    vh = _mm(v, w['m2_wh'])
    vn = _rss(vh, 8)
    ms = _mm(jnp.concatenate([ms, vn], 1), w['m2_ws']) + w['m2_bs']
    v = _mm(vh, w['m2_wv'])
    B = gs.shape[0]
    return jnp.concatenate(
        [ms, v, jnp.ones((B, 1), F32), jnp.zeros((B, 39), F32)], 1)


def _final_math(nf, ag, w):
    s0, v0 = nf[:, 0:64], nf[:, 64:88]
    cnt = jnp.maximum(ag[:, 88:89], 1.0)
    s = s0 + ag[:, 0:64] / cnt
    v = v0 + ag[:, 64:88] / cnt
    s = _ln_s(s, w['n0_g'], w['n0_b'])
    v = _ln_v(v, 8)
    vh = _mm(v, w['f0_wh'])
    vn = _rss(vh, 16)
    fs = _mm(jnp.concatenate([s, vn], 1), w['f0_ws']) + w['f0_bs']
    fv = _mm(vh, w['f0_wv'])
    vh = _mm(fv, w['f1_wh'])
    vn = _rss(vh, 16)
    fs = _mm(jnp.concatenate([fs, vn], 1), w['f1_ws']) + w['f1_bs']
    fv = _mm(vh, w['f1_wv'])
    s = _ln_s(s + fs, w['n1_g'], w['n1_b'])
    v = _ln_v(v + fv, 8)
    s = _ln_s(s, w['fl_g'], w['fl_b'])
    v = _ln_v(v, 8)
    vh = _mm(v, w['o_wh'])
    vn = _rss(vh, 8)
    return _mm(jnp.concatenate([s, vn], 1), w['o_ws']) + w['o_bs']


def _prep_weights(p):
    I3 = jnp.eye(3, dtype=F32)

    def bd(W):
        return jnp.kron(I3, W)

    def row(x):
        return x.reshape(1, -1).astype(F32)

    c = p['convs'][0]
    return {
        'np_wh': bd(p['np_gvp']['Wh']), 'np_ws': p['np_gvp']['Ws'],
        'np_bs': row(p['np_gvp']['bs']), 'np_wv': bd(p['np_gvp']['Wv']),
        'npln_g': row(p['np_ln']['g']), 'npln_b': row(p['np_ln']['b']),
        'ntype_emb': p['ntype_emb'],
        'gnln_g': row(p['gn_ln']['g']), 'gnln_b': row(p['gn_ln']['b']),
        'gn_wh': bd(p['gn_gvp']['Wh']), 'gn_ws': p['gn_gvp']['Ws'],
        'gn_bs': row(p['gn_gvp']['bs']), 'gn_wv': bd(p['gn_gvp']['Wv']),
        'ep_wh': row(p['ep_gvp']['Wh']), 'ep_ws': p['ep_gvp']['Ws'],
        'ep_bs': row(p['ep_gvp']['bs']), 'ep_wv': row(p['ep_gvp']['Wv']),
        'epln_g': row(p['ep_ln']['g']), 'epln_b': row(p['ep_ln']['b']),
        'etype_emb': p['etype_emb'],
        'geln_g': row(p['ge_ln']['g']), 'geln_b': row(p['ge_ln']['b']),
        'ge_wh': row(p['ge_gvp']['Wh']), 'ge_ws': p['ge_gvp']['Ws'],
        'ge_bs': row(p['ge_gvp']['bs']), 'ge_wv': row(p['ge_gvp']['Wv']),
        'm0_wh': bd(c['msg0']['Wh']), 'm0_ws': c['msg0']['Ws'],
        'm0_bs': row(c['msg0']['bs']), 'm0_wv': bd(c['msg0']['Wv']),
        'm1_wh': bd(c['msg1']['Wh']), 'm1_ws': c['msg1']['Ws'],
        'm1_bs': row(c['msg1']['bs']), 'm1_wv': bd(c['msg1']['Wv']),
        'm2_wh': bd(c['msg2']['Wh']), 'm2_ws': c['msg2']['Ws'],
        'm2_bs': row(c['msg2']['bs']), 'm2_wv': bd(c['msg2']['Wv']),
        'n0_g': row(c['norm0']['g']), 'n0_b': row(c['norm0']['b']),
        'f0_wh': bd(c['ff0']['Wh']), 'f0_ws': c['ff0']['Ws'],
        'f0_bs': row(c['ff0']['bs']), 'f0_wv': bd(c['ff0']['Wv']),
        'f1_wh': bd(c['ff1']['Wh']), 'f1_ws': c['ff1']['Ws'],
        'f1_bs': row(c['ff1']['bs']), 'f1_wv': bd(c['ff1']['Wv']),
        'n1_g': row(c['norm1']['g']), 'n1_b': row(c['norm1']['b']),
        'fl_g': row(p['final_ln']['g']), 'fl_b': row(p['final_ln']['b']),
        'o_wh': bd(p['out_gvp']['Wh']), 'o_ws': p['out_gvp']['Ws'],
        'o_bs': row(p['out_gvp']['bs']),
    }


_A_KEYS = ['np_wh', 'np_ws', 'np_bs', 'np_wv', 'npln_g', 'npln_b',
           'ntype_emb', 'gnln_g', 'gnln_b', 'gn_wh', 'gn_ws', 'gn_bs',
           'gn_wv']
_C_KEYS = ['ep_wh', 'ep_ws', 'ep_bs', 'ep_wv', 'epln_g', 'epln_b',
           'etype_emb', 'geln_g', 'geln_b', 'ge_wh', 'ge_ws', 'ge_bs',
           'ge_wv',
           'm0_wh', 'm0_ws', 'm0_bs', 'm0_wv',
           'm1_wh', 'm1_ws', 'm1_bs', 'm1_wv',
           'm2_wh', 'm2_ws', 'm2_bs', 'm2_wv']
_E_KEYS = ['n0_g', 'n0_b', 'f0_wh', 'f0_ws', 'f0_bs', 'f0_wv',
           'f1_wh', 'f1_ws', 'f1_bs', 'f1_wv', 'n1_g', 'n1_b',
           'fl_g', 'fl_b', 'o_wh', 'o_ws', 'o_bs']


def _full_spec(x):
    return pl.BlockSpec(x.shape, lambda i: tuple(0 for _ in x.shape))


def _tc_node_prep(nodes_in, w, n_pad, blk):
    def body(xin_ref, *refs):
        wrefs, out_ref = refs[:-1], refs[-1]
        wd = {k: r[...] for k, r in zip(_A_KEYS, wrefs)}
        out_ref[...] = _node_math(xin_ref[...], wd)

    grid = (n_pad // blk,)
    return pl.pallas_call(
        body,
        grid=grid,
        in_specs=[pl.BlockSpec((blk, 16), lambda i: (i, 0))] +
                 [_full_spec(w[k]) for k in _A_KEYS],
        out_specs=pl.BlockSpec((blk, 128), lambda i: (i, 0)),
        out_shape=jax.ShapeDtypeStruct((n_pad, 128), F32),
    )(nodes_in, *[w[k] for k in _A_KEYS])


def _tc_edge_msgs(gsrc, gdst, ein, w, e_pad, e_real, blk):
    def body(gs_ref, gd_ref, ein_ref, *refs):
        wrefs, out_ref = refs[:-1], refs[-1]
        wd = {k: r[...] for k, r in zip(_C_KEYS, wrefs)}
        m = _edge_math(gs_ref[...], gd_ref[...], ein_ref[...], wd)
        i = pl.program_id(0)
        rows = i * blk + lax.broadcasted_iota(jnp.int32, (blk, 1), 0)
        out_ref[...] = m * (rows < e_real).astype(F32)

    grid = (e_pad // blk,)
    return pl.pallas_call(
        body,
        grid=grid,
        in_specs=[pl.BlockSpec((blk, 128), lambda i: (i, 0)),
                  pl.BlockSpec((blk, 128), lambda i: (i, 0)),
                  pl.BlockSpec((blk, 36), lambda i: (i, 0))] +
                 [_full_spec(w[k]) for k in _C_KEYS],
        out_specs=pl.BlockSpec((blk, 128), lambda i: (i, 0)),
        out_shape=jax.ShapeDtypeStruct((e_pad, 128), F32),
    )(gsrc, gdst, ein, *[w[k] for k in _C_KEYS])


def _tc_final(nf, agg, w, n_pad, blk):
    def body(nf_ref, ag_ref, *refs):
        wrefs, out_ref = refs[:-1], refs[-1]
        wd = {k: r[...] for k, r in zip(_E_KEYS, wrefs)}
        out_ref[...] = _final_math(nf_ref[...], ag_ref[...], wd)

    grid = (n_pad // blk,)
    return pl.pallas_call(
        body,
        grid=grid,
        in_specs=[pl.BlockSpec((blk, 128), lambda i: (i, 0)),
                  pl.BlockSpec((blk, 128), lambda i: (i, 0))] +
                 [_full_spec(w[k]) for k in _E_KEYS],
        out_specs=pl.BlockSpec((blk, 8), lambda i: (i, 0)),
        out_shape=jax.ShapeDtypeStruct((n_pad, 8), F32),
    )(nf, agg, *[w[k] for k in _E_KEYS])


_NC, _NS, _CH = 2, 16, 128


def _sc_gather(nf, src3d, dst3d, e_pad):
    nw = _NC * _NS
    n_chunks = e_pad // (nw * _CH)
    mesh = plsc.VectorSubcoreMesh(core_axis_name="c", subcore_axis_name="s")

    @functools.partial(
        pl.kernel, mesh=mesh,
        out_type=[jax.ShapeDtypeStruct((e_pad, 128), F32),
                  jax.ShapeDtypeStruct((e_pad, 128), F32)],
        scratch_types=[pltpu.VMEM((n_chunks, _CH), jnp.int32),
                       pltpu.VMEM((n_chunks, _CH), jnp.int32),
                       pltpu.VMEM((_CH, 128), F32),
                       pltpu.VMEM((_CH, 128), F32),
                       pltpu.SemaphoreType.DMA,
                       pltpu.SemaphoreType.DMA])
    def k(nf_hbm, src_hbm, dst_hbm, gsrc_hbm, gdst_hbm,
          src_v, dst_v, b0, b1, sem0, sem1):
        wid = lax.axis_index("s") * _NC + lax.axis_index("c")
        c0 = wid * n_chunks
        pltpu.sync_copy(src_hbm.at[wid], src_v)
        pltpu.sync_copy(dst_hbm.at[wid], dst_v)

        def body(j, carry):
            cp0 = pltpu.async_copy(nf_hbm.at[src_v.at[j]], b0, sem0)
            cp1 = pltpu.async_copy(nf_hbm.at[dst_v.at[j]], b1, sem1)
            cp0.wait()
            pltpu.sync_copy(b0, gsrc_hbm.at[pl.ds((c0 + j) * _CH, _CH)])
            cp1.wait()
            pltpu.sync_copy(b1, gdst_hbm.at[pl.ds((c0 + j) * _CH, _CH)])
            return carry

        lax.fori_loop(0, n_chunks, body, 0)

    return k(nf, src3d, dst3d)


def _sc_scatter(msgs, dst3, zrows, n_pad, e_pad):
    chunks_total = e_pad // _CH
    cpt = chunks_total // _NS
    nsub = n_pad // 8
    acc_rows = nsub + 16
    zrt = acc_rows // _NS
    wrt = nsub // _NS
    mesh = plsc.VectorSubcoreMesh(core_axis_name="c", subcore_axis_name="s")

    @functools.partial(
        pl.kernel, mesh=mesh,
        out_type=jax.ShapeDtypeStruct((n_pad, 128), F32),
        scratch_types=[pltpu.VMEM((cpt, 1, _CH), jnp.int32),
                       pltpu.VMEM((1, 1, _CH), jnp.int32),
                       pltpu.VMEM((_CH, 128), F32),
                       pltpu.VMEM_SHARED((acc_rows, 128), F32),
                       pltpu.SemaphoreType.DMA])
    def k(msgs_hbm, dst_hbm, z_hbm, out_hbm, idx_v, ridx, mbuf, acc, sem):
        c = lax.axis_index("c")
        s = lax.axis_index("s")
        pltpu.sync_copy(dst_hbm.at[pl.ds(s * cpt, cpt)], idx_v)
        for kk in range(4):
            r = 4 * c + kk
            base = r * nsub
            pltpu.sync_copy(z_hbm.at[pl.ds(0, zrt)],
                            acc.at[pl.ds(s * zrt, zrt)])
            plsc.subcore_barrier()

            def body(j, carry):
                row0 = (s * cpt + j) * _CH
                cp = pltpu.async_copy(msgs_hbm.at[pl.ds(row0, _CH)],
                                      mbuf, sem)
                for l in range(_CH // 16):
                    iv = idx_v[j, 0, pl.ds(16 * l, 16)]
                    loc = iv - base
                    ok = (loc >= 0) & (loc < nsub)
                    ridx[0, 0, pl.ds(16 * l, 16)] = jnp.where(
                        ok, loc, nsub)
                cp.wait()
                pltpu.sync_copy(mbuf, acc.at[ridx.at[0, 0]], add=True)
                return carry

            lax.fori_loop(0, cpt, body, 0)
            plsc.subcore_barrier()
            pltpu.sync_copy(
                acc.at[pl.ds(s * wrt, wrt)],
                out_hbm.at[pl.ds(base + s * wrt, wrt)])
            plsc.subcore_barrier()

    return k(msgs, dst3, zrows)


def kernel(x_s, x_v, edge_index, ntypes, etypes, eattr_s, eattr_v, params):
    n = x_s.shape[0]
    e = edge_index.shape[1]
    blk_n, blk_e = 512, 1024
    n_pad = -(-n // 256) * 256
    e_pad = -(-e // 4096) * 4096
    w = _prep_weights(params)

    xv_flat = jnp.transpose(x_v, (0, 2, 1)).reshape(n, 9)
    nodes_in = jnp.concatenate(
        [x_s, xv_flat, ntypes.astype(F32)[:, None]], axis=1)
    nodes_in = jnp.pad(nodes_in, ((0, n_pad - n), (0, 0)))

    ein = jnp.concatenate(
        [eattr_s, eattr_v.reshape(e, 3), etypes.astype(F32)[:, None]], axis=1)
    ein = jnp.pad(ein, ((0, e_pad - e), (0, 0)))

    src = jnp.pad(edge_index[0].astype(jnp.int32), (0, e_pad - e))
    dst = jnp.pad(edge_index[1].astype(jnp.int32), (0, e_pad - e))
    nw = _NC * _NS
    src3d = src.reshape(nw, e_pad // (nw * _CH), _CH)
    dst3d = dst.reshape(nw, e_pad // (nw * _CH), _CH)
    dst3 = dst.reshape(e_pad // _CH, 1, _CH)

    nf = _tc_node_prep(nodes_in, w, n_pad, blk_n)
    gsrc, gdst = _sc_gather(nf, src3d, dst3d, e_pad)
    msgs = _tc_edge_msgs(gsrc, gdst, ein, w, e_pad, e, blk_e)
    zrows = jnp.zeros(((n_pad // 8 + 16) // _NS, 128), F32)
    agg = _sc_scatter(msgs, dst3, zrows, n_pad, e_pad)
    out = _tc_final(nf, agg, w, n_pad, blk_n)
    return out[:n]


# speedup vs baseline: 1.0792x; 1.0792x over previous
import functools

import jax
import jax.numpy as jnp
from jax import lax
from jax.experimental import pallas as pl
from jax.experimental.pallas import tpu as pltpu
from jax.experimental.pallas import tpu_sc as plsc

F32 = jnp.float32


def _mm(a, w):
    return lax.dot_general(a, w, (((1,), (0,)), ((), ())),
                           precision=lax.Precision.HIGHEST,
                           preferred_element_type=F32)


def _rss(vh, nch):
    s2 = vh[:, :nch] ** 2 + vh[:, nch:2 * nch] ** 2 + vh[:, 2 * nch:] ** 2
    return jnp.sqrt(jnp.maximum(s2, 1e-8))


def _ln_s(s, g, b):
    mu = jnp.mean(s, axis=-1, keepdims=True)
    var = jnp.mean((s - mu) ** 2, axis=-1, keepdims=True)
    return (s - mu) / jnp.sqrt(var + 1e-5) * g + b


def _ln_v(v, nch):
    s2 = jnp.maximum(v[:, :nch] ** 2 + v[:, nch:2 * nch] ** 2
                     + v[:, 2 * nch:] ** 2, 1e-8)
    rms = jnp.sqrt(jnp.mean(s2, axis=-1, keepdims=True))
    return v / rms


def _node_math(x, w):
    xs, xv, nt = x[:, 0:6], x[:, 6:15], x[:, 15]
    vh = _mm(xv, w['np_wh'])
    vn = _rss(vh, 8)
    s = _mm(jnp.concatenate([xs, vn], 1), w['np_ws']) + w['np_bs']
    v = _mm(vh, w['np_wv'])
    s = _ln_s(s, w['npln_g'], w['npln_b'])
    v = _ln_v(v, 8)
    ids = lax.broadcasted_iota(jnp.int32, (x.shape[0], 20), 1).astype(F32)
    oh = (nt[:, None] == ids).astype(F32)
    s = jnp.concatenate([_mm(oh, w['ntype_emb']), s], 1)
    s = _ln_s(s, w['gnln_g'], w['gnln_b'])
    v = _ln_v(v, 8)
    vh = _mm(v, w['gn_wh'])
    vn = _rss(vh, 8)
    s = _mm(jnp.concatenate([s, vn], 1), w['gn_ws']) + w['gn_bs']
    v = _mm(vh, w['gn_wv'])
    return jnp.concatenate(
        [s, v, jnp.zeros((x.shape[0], 40), F32)], 1)


def _edge_math(gs, gd, ein, w):
    s_src, v_src = gs[:, 0:64], gs[:, 64:88]
    s_dst, v_dst = gd[:, 0:64], gd[:, 64:88]
    es_a, ev, et = ein[:, 0:32], ein[:, 32:35], ein[:, 35]
    vh = ev * w['ep_wh']
    vn = _rss(vh, 1)
    es = _mm(jnp.concatenate([es_a, vn], 1), w['ep_ws']) + w['ep_bs']
    ev1 = vh * w['ep_wv']
    es = _ln_s(es, w['epln_g'], w['epln_b'])
    ev1 = _ln_v(ev1, 1)
    ids = lax.broadcasted_iota(jnp.int32, (ein.shape[0], 4), 1).astype(F32)
    oh = (et[:, None] == ids).astype(F32)
    es = jnp.concatenate([_mm(oh, w['etype_emb']), es], 1)
    es = _ln_s(es, w['geln_g'], w['geln_b'])
    ev1 = _ln_v(ev1, 1)
    vh = ev1 * w['ge_wh']
    vn = _rss(vh, 1)
    es = _mm(jnp.concatenate([es, vn], 1), w['ge_ws']) + w['ge_bs']
    ev2 = vh * w['ge_wv']
    ms = jnp.concatenate([s_src, es, s_dst], 1)
    mv = jnp.concatenate([
        v_src[:, 0:8], ev2[:, 0:1], v_dst[:, 0:8],
        v_src[:, 8:16], ev2[:, 1:2], v_dst[:, 8:16],
        v_src[:, 16:24], ev2[:, 2:3], v_dst[:, 16:24]], 1)
    vh = _mm(mv, w['m0_wh'])
    vn = _rss(vh, 17)
    ms = _mm(jnp.concatenate([ms, vn], 1), w['m0_ws']) + w['m0_bs']
    v = _mm(vh, w['m0_wv'])
    vh = _mm(v, w['m1_wh'])
    vn = _rss(vh, 8)
    ms = _mm(jnp.concatenate([ms, vn], 1), w['m1_ws']) + w['m1_bs']
    v = _mm(vh, w['m1_wv'])
    vh = _mm(v, w['m2_wh'])
    vn = _rss(vh, 8)
    ms = _mm(jnp.concatenate([ms, vn], 1), w['m2_ws']) + w['m2_bs']
    v = _mm(vh, w['m2_wv'])
    B = gs.shape[0]
    return jnp.concatenate(
        [ms, v, jnp.ones((B, 1), F32), jnp.zeros((B, 39), F32)], 1)


def _final_math(nf, ag, w):
    s0, v0 = nf[:, 0:64], nf[:, 64:88]
    cnt = jnp.maximum(ag[:, 88:89], 1.0)
    s = s0 + ag[:, 0:64] / cnt
    v = v0 + ag[:, 64:88] / cnt
    s = _ln_s(s, w['n0_g'], w['n0_b'])
    v = _ln_v(v, 8)
    vh = _mm(v, w['f0_wh'])
    vn = _rss(vh, 16)
    fs = _mm(jnp.concatenate([s, vn], 1), w['f0_ws']) + w['f0_bs']
    fv = _mm(vh, w['f0_wv'])
    vh = _mm(fv, w['f1_wh'])
    vn = _rss(vh, 16)
    fs = _mm(jnp.concatenate([fs, vn], 1), w['f1_ws']) + w['f1_bs']
    fv = _mm(vh, w['f1_wv'])
    s = _ln_s(s + fs, w['n1_g'], w['n1_b'])
    v = _ln_v(v + fv, 8)
    s = _ln_s(s, w['fl_g'], w['fl_b'])
    v = _ln_v(v, 8)
    vh = _mm(v, w['o_wh'])
    vn = _rss(vh, 8)
    return _mm(jnp.concatenate([s, vn], 1), w['o_ws']) + w['o_bs']


def _prep_weights(p):
    I3 = jnp.eye(3, dtype=F32)

    def bd(W):
        return jnp.kron(I3, W)

    def row(x):
        return x.reshape(1, -1).astype(F32)

    c = p['convs'][0]
    return {
        'np_wh': bd(p['np_gvp']['Wh']), 'np_ws': p['np_gvp']['Ws'],
        'np_bs': row(p['np_gvp']['bs']), 'np_wv': bd(p['np_gvp']['Wv']),
        'npln_g': row(p['np_ln']['g']), 'npln_b': row(p['np_ln']['b']),
        'ntype_emb': p['ntype_emb'],
        'gnln_g': row(p['gn_ln']['g']), 'gnln_b': row(p['gn_ln']['b']),
        'gn_wh': bd(p['gn_gvp']['Wh']), 'gn_ws': p['gn_gvp']['Ws'],
        'gn_bs': row(p['gn_gvp']['bs']), 'gn_wv': bd(p['gn_gvp']['Wv']),
        'ep_wh': row(p['ep_gvp']['Wh']), 'ep_ws': p['ep_gvp']['Ws'],
        'ep_bs': row(p['ep_gvp']['bs']), 'ep_wv': row(p['ep_gvp']['Wv']),
        'epln_g': row(p['ep_ln']['g']), 'epln_b': row(p['ep_ln']['b']),
        'etype_emb': p['etype_emb'],
        'geln_g': row(p['ge_ln']['g']), 'geln_b': row(p['ge_ln']['b']),
        'ge_wh': row(p['ge_gvp']['Wh']), 'ge_ws': p['ge_gvp']['Ws'],
        'ge_bs': row(p['ge_gvp']['bs']), 'ge_wv': row(p['ge_gvp']['Wv']),
        'm0_wh': bd(c['msg0']['Wh']), 'm0_ws': c['msg0']['Ws'],
        'm0_bs': row(c['msg0']['bs']), 'm0_wv': bd(c['msg0']['Wv']),
        'm1_wh': bd(c['msg1']['Wh']), 'm1_ws': c['msg1']['Ws'],
        'm1_bs': row(c['msg1']['bs']), 'm1_wv': bd(c['msg1']['Wv']),
        'm2_wh': bd(c['msg2']['Wh']), 'm2_ws': c['msg2']['Ws'],
        'm2_bs': row(c['msg2']['bs']), 'm2_wv': bd(c['msg2']['Wv']),
        'n0_g': row(c['norm0']['g']), 'n0_b': row(c['norm0']['b']),
        'f0_wh': bd(c['ff0']['Wh']), 'f0_ws': c['ff0']['Ws'],
        'f0_bs': row(c['ff0']['bs']), 'f0_wv': bd(c['ff0']['Wv']),
        'f1_wh': bd(c['ff1']['Wh']), 'f1_ws': c['ff1']['Ws'],
        'f1_bs': row(c['ff1']['bs']), 'f1_wv': bd(c['ff1']['Wv']),
        'n1_g': row(c['norm1']['g']), 'n1_b': row(c['norm1']['b']),
        'fl_g': row(p['final_ln']['g']), 'fl_b': row(p['final_ln']['b']),
        'o_wh': bd(p['out_gvp']['Wh']), 'o_ws': p['out_gvp']['Ws'],
        'o_bs': row(p['out_gvp']['bs']),
    }


_A_KEYS = ['np_wh', 'np_ws', 'np_bs', 'np_wv', 'npln_g', 'npln_b',
           'ntype_emb', 'gnln_g', 'gnln_b', 'gn_wh', 'gn_ws', 'gn_bs',
           'gn_wv']
_C_KEYS = ['ep_wh', 'ep_ws', 'ep_bs', 'ep_wv', 'epln_g', 'epln_b',
           'etype_emb', 'geln_g', 'geln_b', 'ge_wh', 'ge_ws', 'ge_bs',
           'ge_wv',
           'm0_wh', 'm0_ws', 'm0_bs', 'm0_wv',
           'm1_wh', 'm1_ws', 'm1_bs', 'm1_wv',
           'm2_wh', 'm2_ws', 'm2_bs', 'm2_wv']
_E_KEYS = ['n0_g', 'n0_b', 'f0_wh', 'f0_ws', 'f0_bs', 'f0_wv',
           'f1_wh', 'f1_ws', 'f1_bs', 'f1_wv', 'n1_g', 'n1_b',
           'fl_g', 'fl_b', 'o_wh', 'o_ws', 'o_bs']


def _full_spec(x):
    return pl.BlockSpec(x.shape, lambda i: tuple(0 for _ in x.shape))


def _tc_node_prep(nodes_in, w, n_pad, blk):
    def body(xin_ref, *refs):
        wrefs, out_ref = refs[:-1], refs[-1]
        wd = {k: r[...] for k, r in zip(_A_KEYS, wrefs)}
        out_ref[...] = _node_math(xin_ref[...], wd)

    grid = (n_pad // blk,)
    return pl.pallas_call(
        body,
        grid=grid,
        in_specs=[pl.BlockSpec((blk, 16), lambda i: (i, 0))] +
                 [_full_spec(w[k]) for k in _A_KEYS],
        out_specs=pl.BlockSpec((blk, 128), lambda i: (i, 0)),
        out_shape=jax.ShapeDtypeStruct((n_pad, 128), F32),
    )(nodes_in, *[w[k] for k in _A_KEYS])


def _tc_edge_msgs(gsrc, gdst, ein, w, e_pad, e_real, blk):
    def body(gs_ref, gd_ref, ein_ref, *refs):
        wrefs, out_ref = refs[:-1], refs[-1]
        wd = {k: r[...] for k, r in zip(_C_KEYS, wrefs)}
        m = _edge_math(gs_ref[...], gd_ref[...], ein_ref[...], wd)
        i = pl.program_id(0)
        rows = i * blk + lax.broadcasted_iota(jnp.int32, (blk, 1), 0)
        out_ref[...] = m * (rows < e_real).astype(F32)

    grid = (e_pad // blk,)
    return pl.pallas_call(
        body,
        grid=grid,
        in_specs=[pl.BlockSpec((blk, 128), lambda i: (i, 0)),
                  pl.BlockSpec((blk, 128), lambda i: (i, 0)),
                  pl.BlockSpec((blk, 36), lambda i: (i, 0))] +
                 [_full_spec(w[k]) for k in _C_KEYS],
        out_specs=pl.BlockSpec((blk, 128), lambda i: (i, 0)),
        out_shape=jax.ShapeDtypeStruct((e_pad, 128), F32),
    )(gsrc, gdst, ein, *[w[k] for k in _C_KEYS])


def _tc_final(nf, agg, w, n_pad, blk):
    def body(nf_ref, ag_ref, *refs):
        wrefs, out_ref = refs[:-1], refs[-1]
        wd = {k: r[...] for k, r in zip(_E_KEYS, wrefs)}
        out_ref[...] = _final_math(nf_ref[...], ag_ref[...], wd)

    grid = (n_pad // blk,)
    return pl.pallas_call(
        body,
        grid=grid,
        in_specs=[pl.BlockSpec((blk, 128), lambda i: (i, 0)),
                  pl.BlockSpec((blk, 128), lambda i: (i, 0))] +
                 [_full_spec(w[k]) for k in _E_KEYS],
        out_specs=pl.BlockSpec((blk, 8), lambda i: (i, 0)),
        out_shape=jax.ShapeDtypeStruct((n_pad, 8), F32),
    )(nf, agg, *[w[k] for k in _E_KEYS])


_NC, _NS, _CH = 2, 16, 128


def _sc_gather(nf, src3d, dst3d, e_pad):
    nw = _NC * _NS
    n_chunks = e_pad // (nw * _CH)
    mesh = plsc.VectorSubcoreMesh(core_axis_name="c", subcore_axis_name="s")

    @functools.partial(
        pl.kernel, mesh=mesh,
        out_type=[jax.ShapeDtypeStruct((e_pad, 128), F32),
                  jax.ShapeDtypeStruct((e_pad, 128), F32)],
        scratch_types=[pltpu.VMEM((n_chunks, _CH), jnp.int32),
                       pltpu.VMEM((n_chunks, _CH), jnp.int32),
                       pltpu.VMEM((_CH, 128), F32),
                       pltpu.VMEM((_CH, 128), F32),
                       pltpu.SemaphoreType.DMA,
                       pltpu.SemaphoreType.DMA])
    def k(nf_hbm, src_hbm, dst_hbm, gsrc_hbm, gdst_hbm,
          src_v, dst_v, b0, b1, sem0, sem1):
        wid = lax.axis_index("s") * _NC + lax.axis_index("c")
        c0 = wid * n_chunks
        pltpu.sync_copy(src_hbm.at[wid], src_v)
        pltpu.sync_copy(dst_hbm.at[wid], dst_v)

        def body(j, carry):
            cp0 = pltpu.async_copy(nf_hbm.at[src_v.at[j]], b0, sem0)
            cp1 = pltpu.async_copy(nf_hbm.at[dst_v.at[j]], b1, sem1)
            cp0.wait()
            pltpu.sync_copy(b0, gsrc_hbm.at[pl.ds((c0 + j) * _CH, _CH)])
            cp1.wait()
            pltpu.sync_copy(b1, gdst_hbm.at[pl.ds((c0 + j) * _CH, _CH)])
            return carry

        lax.fori_loop(0, n_chunks, body, 0)

    return k(nf, src3d, dst3d)


def _sc_scatter(msgs, dst3, zrows, n_pad, e_pad):
    chunks_total = e_pad // _CH
    cpt = chunks_total // _NS
    nsub = n_pad // 8
    acc_rows = nsub + 16
    zrt = acc_rows // _NS
    wrt = nsub // _NS
    mesh = plsc.VectorSubcoreMesh(core_axis_name="c", subcore_axis_name="s")

    @functools.partial(
        pl.kernel, mesh=mesh,
        out_type=jax.ShapeDtypeStruct((n_pad, 128), F32),
        scratch_types=[pltpu.VMEM((2, 1, _CH), jnp.int32),
                       pltpu.VMEM((2, 1, _CH), jnp.int32),
                       pltpu.VMEM((_CH, 128), F32),
                       pltpu.VMEM((_CH, 128), F32),
                       pltpu.VMEM_SHARED((acc_rows, 128), F32),
                       pltpu.SemaphoreType.DMA,
                       pltpu.SemaphoreType.DMA,
                       pltpu.SemaphoreType.DMA,
                       pltpu.SemaphoreType.DMA])
    def k(msgs_hbm, dst_hbm, z_hbm, out_hbm, idxb, ridx, mb0, mb1,
          acc, sm0, sm1, si0, si1):
        c = lax.axis_index("c")
        s = lax.axis_index("s")
        bufs = ((mb0, sm0, si0, 0), (mb1, sm1, si1, 1))

        def _load(j, slot):
            buf, sm, si, sl = bufs[slot]
            row0 = (s * cpt + j) * _CH
            pltpu.async_copy(msgs_hbm.at[pl.ds(row0, _CH)], buf, sm)
            pltpu.async_copy(dst_hbm.at[pl.ds(s * cpt + j, 1)],
                             idxb.at[pl.ds(sl, 1)], si)

        def _drain(slot):
            buf, sm, si, sl = bufs[slot]
            pltpu.make_async_copy(msgs_hbm.at[pl.ds(0, _CH)], buf,
                                  sm).wait()
            pltpu.make_async_copy(dst_hbm.at[pl.ds(0, 1)],
                                  idxb.at[pl.ds(sl, 1)], si).wait()

        for kk in range(4):
            r = 4 * c + kk
            base = r * nsub
            pltpu.sync_copy(z_hbm.at[pl.ds(0, zrt)],
                            acc.at[pl.ds(s * zrt, zrt)])
            plsc.subcore_barrier()

            def _scat(slot):
                sl = bufs[slot][3]
                _drain(slot)
                for l in range(_CH // 16):
                    iv = idxb[sl, 0, pl.ds(16 * l, 16)]
                    loc = iv - base
                    ok = (loc >= 0) & (loc < nsub)
                    ridx[sl, 0, pl.ds(16 * l, 16)] = jnp.where(
                        ok, loc, nsub)
                pltpu.sync_copy(bufs[slot][0],
                                acc.at[ridx.at[sl, 0]], add=True)

            _load(0, 0)

            def body(j2, carry):
                a = 2 * j2
                _load(a + 1, 1)
                _scat(0)
                _load(jnp.minimum(a + 2, cpt - 1), 0)
                _scat(1)
                return carry

            lax.fori_loop(0, cpt // 2, body, 0)
            _drain(0)
            plsc.subcore_barrier()
            pltpu.sync_copy(
                acc.at[pl.ds(s * wrt, wrt)],
                out_hbm.at[pl.ds(base + s * wrt, wrt)])
            plsc.subcore_barrier()

    return k(msgs, dst3, zrows)


def kernel(x_s, x_v, edge_index, ntypes, etypes, eattr_s, eattr_v, params):
    n = x_s.shape[0]
    e = edge_index.shape[1]
    blk_n, blk_e = 256, 512
    n_pad = -(-n // 256) * 256
    e_pad = -(-e // 4096) * 4096
    w = _prep_weights(params)

    xv_flat = jnp.transpose(x_v, (0, 2, 1)).reshape(n, 9)
    nodes_in = jnp.concatenate(
        [x_s, xv_flat, ntypes.astype(F32)[:, None]], axis=1)
    nodes_in = jnp.pad(nodes_in, ((0, n_pad - n), (0, 0)))

    ein = jnp.concatenate(
        [eattr_s, eattr_v.reshape(e, 3), etypes.astype(F32)[:, None]], axis=1)
    ein = jnp.pad(ein, ((0, e_pad - e), (0, 0)))

    src = jnp.pad(edge_index[0].astype(jnp.int32), (0, e_pad - e))
    dst = jnp.pad(edge_index[1].astype(jnp.int32), (0, e_pad - e))
    nw = _NC * _NS
    src3d = src.reshape(nw, e_pad // (nw * _CH), _CH)
    dst3d = dst.reshape(nw, e_pad // (nw * _CH), _CH)
    dst3 = dst.reshape(e_pad // _CH, 1, _CH)

    nf = _tc_node_prep(nodes_in, w, n_pad, blk_n)
    gsrc, gdst = _sc_gather(nf, src3d, dst3d, e_pad)
    msgs = _tc_edge_msgs(gsrc, gdst, ein, w, e_pad, e, blk_e)
    zrows = jnp.zeros(((n_pad // 8 + 16) // _NS, 128), F32)
    agg = _sc_scatter(msgs, dst3, zrows, n_pad, e_pad)
    out = _tc_final(nf, agg, w, n_pad, blk_n)
    return out[:n]


# speedup vs baseline: 1.1807x; 1.0941x over previous
import functools

import jax
import jax.numpy as jnp
from jax import lax
from jax.experimental import pallas as pl
from jax.experimental.pallas import tpu as pltpu
from jax.experimental.pallas import tpu_sc as plsc

F32 = jnp.float32


def _mm(a, w):
    return lax.dot_general(a, w, (((1,), (0,)), ((), ())),
                           precision=lax.Precision.HIGHEST,
                           preferred_element_type=F32)


def _rss(vh, nch):
    s2 = vh[:, :nch] ** 2 + vh[:, nch:2 * nch] ** 2 + vh[:, 2 * nch:] ** 2
    return jnp.sqrt(jnp.maximum(s2, 1e-8))


def _ln_s(s, g, b):
    mu = jnp.mean(s, axis=-1, keepdims=True)
    var = jnp.mean((s - mu) ** 2, axis=-1, keepdims=True)
    return (s - mu) / jnp.sqrt(var + 1e-5) * g + b


def _ln_v(v, nch):
    s2 = jnp.maximum(v[:, :nch] ** 2 + v[:, nch:2 * nch] ** 2
                     + v[:, 2 * nch:] ** 2, 1e-8)
    rms = jnp.sqrt(jnp.mean(s2, axis=-1, keepdims=True))
    return v / rms


def _node_math(x, w):
    xs, xv, nt = x[:, 0:6], x[:, 6:15], x[:, 15]
    vh = _mm(xv, w['np_wh'])
    vn = _rss(vh, 8)
    s = _mm(jnp.concatenate([xs, vn], 1), w['np_ws']) + w['np_bs']
    v = _mm(vh, w['np_wv'])
    s = _ln_s(s, w['npln_g'], w['npln_b'])
    v = _ln_v(v, 8)
    ids = lax.broadcasted_iota(jnp.int32, (x.shape[0], 20), 1).astype(F32)
    oh = (nt[:, None] == ids).astype(F32)
    s = jnp.concatenate([_mm(oh, w['ntype_emb']), s], 1)
    s = _ln_s(s, w['gnln_g'], w['gnln_b'])
    v = _ln_v(v, 8)
    vh = _mm(v, w['gn_wh'])
    vn = _rss(vh, 8)
    s = _mm(jnp.concatenate([s, vn], 1), w['gn_ws']) + w['gn_bs']
    v = _mm(vh, w['gn_wv'])
    return jnp.concatenate(
        [s, v, jnp.zeros((x.shape[0], 40), F32)], 1)


def _edge_math(gs, gd, ein, w):
    s_src, v_src = gs[:, 0:64], gs[:, 64:88]
    s_dst, v_dst = gd[:, 0:64], gd[:, 64:88]
    es_a, ev, et = ein[:, 0:32], ein[:, 32:35], ein[:, 35]
    vh = ev * w['ep_wh']
    vn = _rss(vh, 1)
    es = _mm(jnp.concatenate([es_a, vn], 1), w['ep_ws']) + w['ep_bs']
    ev1 = vh * w['ep_wv']
    es = _ln_s(es, w['epln_g'], w['epln_b'])
    ev1 = _ln_v(ev1, 1)
    ids = lax.broadcasted_iota(jnp.int32, (ein.shape[0], 4), 1).astype(F32)
    oh = (et[:, None] == ids).astype(F32)
    es = jnp.concatenate([_mm(oh, w['etype_emb']), es], 1)
    es = _ln_s(es, w['geln_g'], w['geln_b'])
    ev1 = _ln_v(ev1, 1)
    vh = ev1 * w['ge_wh']
    vn = _rss(vh, 1)
    es = _mm(jnp.concatenate([es, vn], 1), w['ge_ws']) + w['ge_bs']
    ev2 = vh * w['ge_wv']
    ms = jnp.concatenate([s_src, es, s_dst], 1)
    mv = jnp.concatenate([
        v_src[:, 0:8], ev2[:, 0:1], v_dst[:, 0:8],
        v_src[:, 8:16], ev2[:, 1:2], v_dst[:, 8:16],
        v_src[:, 16:24], ev2[:, 2:3], v_dst[:, 16:24]], 1)
    vh = _mm(mv, w['m0_wh'])
    vn = _rss(vh, 17)
    ms = _mm(jnp.concatenate([ms, vn], 1), w['m0_ws']) + w['m0_bs']
    v = _mm(vh, w['m0_wv'])
    vh = _mm(v, w['m1_wh'])
    vn = _rss(vh, 8)
    ms = _mm(jnp.concatenate([ms, vn], 1), w['m1_ws']) + w['m1_bs']
    v = _mm(vh, w['m1_wv'])
    vh = _mm(v, w['m2_wh'])
    vn = _rss(vh, 8)
    ms = _mm(jnp.concatenate([ms, vn], 1), w['m2_ws']) + w['m2_bs']
    v = _mm(vh, w['m2_wv'])
    B = gs.shape[0]
    return jnp.concatenate(
        [ms, v, jnp.ones((B, 1), F32), jnp.zeros((B, 39), F32)], 1)


def _final_math(nf, ag, w):
    s0, v0 = nf[:, 0:64], nf[:, 64:88]
    cnt = jnp.maximum(ag[:, 88:89], 1.0)
    s = s0 + ag[:, 0:64] / cnt
    v = v0 + ag[:, 64:88] / cnt
    s = _ln_s(s, w['n0_g'], w['n0_b'])
    v = _ln_v(v, 8)
    vh = _mm(v, w['f0_wh'])
    vn = _rss(vh, 16)
    fs = _mm(jnp.concatenate([s, vn], 1), w['f0_ws']) + w['f0_bs']
    fv = _mm(vh, w['f0_wv'])
    vh = _mm(fv, w['f1_wh'])
    vn = _rss(vh, 16)
    fs = _mm(jnp.concatenate([fs, vn], 1), w['f1_ws']) + w['f1_bs']
    fv = _mm(vh, w['f1_wv'])
    s = _ln_s(s + fs, w['n1_g'], w['n1_b'])
    v = _ln_v(v + fv, 8)
    s = _ln_s(s, w['fl_g'], w['fl_b'])
    v = _ln_v(v, 8)
    vh = _mm(v, w['o_wh'])
    vn = _rss(vh, 8)
    return _mm(jnp.concatenate([s, vn], 1), w['o_ws']) + w['o_bs']


def _prep_weights(p):
    I3 = jnp.eye(3, dtype=F32)

    def bd(W):
        return jnp.kron(I3, W)

    def row(x):
        return x.reshape(1, -1).astype(F32)

    c = p['convs'][0]
    return {
        'np_wh': bd(p['np_gvp']['Wh']), 'np_ws': p['np_gvp']['Ws'],
        'np_bs': row(p['np_gvp']['bs']), 'np_wv': bd(p['np_gvp']['Wv']),
        'npln_g': row(p['np_ln']['g']), 'npln_b': row(p['np_ln']['b']),
        'ntype_emb': p['ntype_emb'],
        'gnln_g': row(p['gn_ln']['g']), 'gnln_b': row(p['gn_ln']['b']),
        'gn_wh': bd(p['gn_gvp']['Wh']), 'gn_ws': p['gn_gvp']['Ws'],
        'gn_bs': row(p['gn_gvp']['bs']), 'gn_wv': bd(p['gn_gvp']['Wv']),
        'ep_wh': row(p['ep_gvp']['Wh']), 'ep_ws': p['ep_gvp']['Ws'],
        'ep_bs': row(p['ep_gvp']['bs']), 'ep_wv': row(p['ep_gvp']['Wv']),
        'epln_g': row(p['ep_ln']['g']), 'epln_b': row(p['ep_ln']['b']),
        'etype_emb': p['etype_emb'],
        'geln_g': row(p['ge_ln']['g']), 'geln_b': row(p['ge_ln']['b']),
        'ge_wh': row(p['ge_gvp']['Wh']), 'ge_ws': p['ge_gvp']['Ws'],
        'ge_bs': row(p['ge_gvp']['bs']), 'ge_wv': row(p['ge_gvp']['Wv']),
        'm0_wh': bd(c['msg0']['Wh']), 'm0_ws': c['msg0']['Ws'],
        'm0_bs': row(c['msg0']['bs']), 'm0_wv': bd(c['msg0']['Wv']),
        'm1_wh': bd(c['msg1']['Wh']), 'm1_ws': c['msg1']['Ws'],
        'm1_bs': row(c['msg1']['bs']), 'm1_wv': bd(c['msg1']['Wv']),
        'm2_wh': bd(c['msg2']['Wh']), 'm2_ws': c['msg2']['Ws'],
        'm2_bs': row(c['msg2']['bs']), 'm2_wv': bd(c['msg2']['Wv']),
        'n0_g': row(c['norm0']['g']), 'n0_b': row(c['norm0']['b']),
        'f0_wh': bd(c['ff0']['Wh']), 'f0_ws': c['ff0']['Ws'],
        'f0_bs': row(c['ff0']['bs']), 'f0_wv': bd(c['ff0']['Wv']),
        'f1_wh': bd(c['ff1']['Wh']), 'f1_ws': c['ff1']['Ws'],
        'f1_bs': row(c['ff1']['bs']), 'f1_wv': bd(c['ff1']['Wv']),
        'n1_g': row(c['norm1']['g']), 'n1_b': row(c['norm1']['b']),
        'fl_g': row(p['final_ln']['g']), 'fl_b': row(p['final_ln']['b']),
        'o_wh': bd(p['out_gvp']['Wh']), 'o_ws': p['out_gvp']['Ws'],
        'o_bs': row(p['out_gvp']['bs']),
    }


_A_KEYS = ['np_wh', 'np_ws', 'np_bs', 'np_wv', 'npln_g', 'npln_b',
           'ntype_emb', 'gnln_g', 'gnln_b', 'gn_wh', 'gn_ws', 'gn_bs',
           'gn_wv']
_C_KEYS = ['ep_wh', 'ep_ws', 'ep_bs', 'ep_wv', 'epln_g', 'epln_b',
           'etype_emb', 'geln_g', 'geln_b', 'ge_wh', 'ge_ws', 'ge_bs',
           'ge_wv',
           'm0_wh', 'm0_ws', 'm0_bs', 'm0_wv',
           'm1_wh', 'm1_ws', 'm1_bs', 'm1_wv',
           'm2_wh', 'm2_ws', 'm2_bs', 'm2_wv']
_E_KEYS = ['n0_g', 'n0_b', 'f0_wh', 'f0_ws', 'f0_bs', 'f0_wv',
           'f1_wh', 'f1_ws', 'f1_bs', 'f1_wv', 'n1_g', 'n1_b',
           'fl_g', 'fl_b', 'o_wh', 'o_ws', 'o_bs']


def _full_spec(x):
    return pl.BlockSpec(x.shape, lambda i: tuple(0 for _ in x.shape))


def _tc_node_prep(nodes_in, w, n_pad, blk):
    def body(xin_ref, *refs):
        wrefs, out_ref = refs[:-1], refs[-1]
        wd = {k: r[...] for k, r in zip(_A_KEYS, wrefs)}
        out_ref[...] = _node_math(xin_ref[...], wd)

    grid = (n_pad // blk,)
    return pl.pallas_call(
        body,
        grid=grid,
        in_specs=[pl.BlockSpec((blk, 16), lambda i: (i, 0))] +
                 [_full_spec(w[k]) for k in _A_KEYS],
        out_specs=pl.BlockSpec((blk, 128), lambda i: (i, 0)),
        out_shape=jax.ShapeDtypeStruct((n_pad, 128), F32),
    )(nodes_in, *[w[k] for k in _A_KEYS])


def _tc_edge_msgs(gsrc, gdst, ein, w, e_pad, e_real, blk):
    def body(gs_ref, gd_ref, ein_ref, *refs):
        wrefs, out_ref = refs[:-1], refs[-1]
        wd = {k: r[...] for k, r in zip(_C_KEYS, wrefs)}
        m = _edge_math(gs_ref[...], gd_ref[...], ein_ref[...], wd)
        i = pl.program_id(0)
        rows = i * blk + lax.broadcasted_iota(jnp.int32, (blk, 1), 0)
        out_ref[...] = m * (rows < e_real).astype(F32)

    grid = (e_pad // blk,)
    return pl.pallas_call(
        body,
        grid=grid,
        in_specs=[pl.BlockSpec((blk, 128), lambda i: (i, 0)),
                  pl.BlockSpec((blk, 128), lambda i: (i, 0)),
                  pl.BlockSpec((blk, 36), lambda i: (i, 0))] +
                 [_full_spec(w[k]) for k in _C_KEYS],
        out_specs=pl.BlockSpec((blk, 128), lambda i: (i, 0)),
        out_shape=jax.ShapeDtypeStruct((e_pad, 128), F32),
    )(gsrc, gdst, ein, *[w[k] for k in _C_KEYS])


def _tc_final(nf, agg, w, n_pad, blk):
    def body(nf_ref, ag_ref, *refs):
        wrefs, out_ref = refs[:-1], refs[-1]
        wd = {k: r[...] for k, r in zip(_E_KEYS, wrefs)}
        out_ref[...] = _final_math(nf_ref[...], ag_ref[...], wd)

    grid = (n_pad // blk,)
    return pl.pallas_call(
        body,
        grid=grid,
        in_specs=[pl.BlockSpec((blk, 128), lambda i: (i, 0)),
                  pl.BlockSpec((blk, 128), lambda i: (i, 0))] +
                 [_full_spec(w[k]) for k in _E_KEYS],
        out_specs=pl.BlockSpec((blk, 8), lambda i: (i, 0)),
        out_shape=jax.ShapeDtypeStruct((n_pad, 8), F32),
    )(nf, agg, *[w[k] for k in _E_KEYS])


_NC, _NS, _CH = 2, 16, 128


def _sc_gather(nf, src3d, dst3d, e_pad):
    nw = _NC * _NS
    n_chunks = e_pad // (nw * _CH)
    mesh = plsc.VectorSubcoreMesh(core_axis_name="c", subcore_axis_name="s")

    @functools.partial(
        pl.kernel, mesh=mesh,
        out_type=[jax.ShapeDtypeStruct((e_pad, 128), F32),
                  jax.ShapeDtypeStruct((e_pad, 128), F32)],
        scratch_types=[pltpu.VMEM((n_chunks, _CH), jnp.int32),
                       pltpu.VMEM((n_chunks, _CH), jnp.int32),
                       pltpu.VMEM((_CH, 128), F32),
                       pltpu.VMEM((_CH, 128), F32),
                       pltpu.SemaphoreType.DMA,
                       pltpu.SemaphoreType.DMA])
    def k(nf_hbm, src_hbm, dst_hbm, gsrc_hbm, gdst_hbm,
          src_v, dst_v, b0, b1, sem0, sem1):
        wid = lax.axis_index("s") * _NC + lax.axis_index("c")
        c0 = wid * n_chunks
        pltpu.sync_copy(src_hbm.at[wid], src_v)
        pltpu.sync_copy(dst_hbm.at[wid], dst_v)

        def body(j, carry):
            cp0 = pltpu.async_copy(nf_hbm.at[src_v.at[j]], b0, sem0)
            cp1 = pltpu.async_copy(nf_hbm.at[dst_v.at[j]], b1, sem1)
            cp0.wait()
            pltpu.sync_copy(b0, gsrc_hbm.at[pl.ds((c0 + j) * _CH, _CH)])
            cp1.wait()
            pltpu.sync_copy(b1, gdst_hbm.at[pl.ds((c0 + j) * _CH, _CH)])
            return carry

        lax.fori_loop(0, n_chunks, body, 0)

    return k(nf, src3d, dst3d)


def _sc_scatter(msgs, dst3, zrows, n_pad, e_pad):
    chunks_total = e_pad // _CH
    cpt = chunks_total // _NS
    nsub = n_pad // 4
    acc_rows = nsub + 16
    zrt = acc_rows // _NS
    wrt = nsub // _NS
    mesh = plsc.VectorSubcoreMesh(core_axis_name="c", subcore_axis_name="s")

    @functools.partial(
        pl.kernel, mesh=mesh,
        out_type=jax.ShapeDtypeStruct((n_pad, 128), F32),
        scratch_types=[pltpu.VMEM((1, 1, _CH), jnp.int32),
                       pltpu.VMEM((1, 1, _CH), jnp.int32),
                       pltpu.VMEM((_CH, 128), F32),
                       pltpu.VMEM_SHARED((acc_rows, 128), F32),
                       pltpu.SemaphoreType.DMA,
                       pltpu.SemaphoreType.DMA])
    def k(msgs_hbm, dst_hbm, z_hbm, out_hbm, idxb, ridx, mb0, acc, sm0, si0):
        c = lax.axis_index("c")
        s = lax.axis_index("s")

        for kk in range(2):
            r = 2 * c + kk
            base = r * nsub
            pltpu.sync_copy(z_hbm.at[pl.ds(0, zrt)],
                            acc.at[pl.ds(s * zrt, zrt)])
            plsc.subcore_barrier()

            def body(j, carry):
                row0 = (s * cpt + j) * _CH
                cp = pltpu.async_copy(msgs_hbm.at[pl.ds(row0, _CH)],
                                      mb0, sm0)
                ci = pltpu.async_copy(dst_hbm.at[pl.ds(s * cpt + j, 1)],
                                      idxb, si0)
                ci.wait()
                for l in range(_CH // 16):
                    iv = idxb[0, 0, pl.ds(16 * l, 16)]
                    loc = iv - base
                    ok = (loc >= 0) & (loc < nsub)
                    ridx[0, 0, pl.ds(16 * l, 16)] = jnp.where(
                        ok, loc, nsub)
                cp.wait()
                pltpu.sync_copy(mb0, acc.at[ridx.at[0, 0]], add=True)
                return carry

            lax.fori_loop(0, cpt, body, 0)
            plsc.subcore_barrier()
            pltpu.sync_copy(
                acc.at[pl.ds(s * wrt, wrt)],
                out_hbm.at[pl.ds(base + s * wrt, wrt)])
            plsc.subcore_barrier()

    return k(msgs, dst3, zrows)


def kernel(x_s, x_v, edge_index, ntypes, etypes, eattr_s, eattr_v, params):
    n = x_s.shape[0]
    e = edge_index.shape[1]
    blk_n, blk_e = 256, 512
    n_pad = -(-n // 256) * 256
    e_pad = -(-e // 4096) * 4096
    w = _prep_weights(params)

    xv_flat = jnp.transpose(x_v, (0, 2, 1)).reshape(n, 9)
    nodes_in = jnp.concatenate(
        [x_s, xv_flat, ntypes.astype(F32)[:, None]], axis=1)
    nodes_in = jnp.pad(nodes_in, ((0, n_pad - n), (0, 0)))

    ein = jnp.concatenate(
        [eattr_s, eattr_v.reshape(e, 3), etypes.astype(F32)[:, None]], axis=1)
    ein = jnp.pad(ein, ((0, e_pad - e), (0, 0)))

    src = jnp.pad(edge_index[0].astype(jnp.int32), (0, e_pad - e))
    dst = jnp.pad(edge_index[1].astype(jnp.int32), (0, e_pad - e))
    nw = _NC * _NS
    src3d = src.reshape(nw, e_pad // (nw * _CH), _CH)
    dst3d = dst.reshape(nw, e_pad // (nw * _CH), _CH)
    dst3 = dst.reshape(e_pad // _CH, 1, _CH)

    nf = _tc_node_prep(nodes_in, w, n_pad, blk_n)
    gsrc, gdst = _sc_gather(nf, src3d, dst3d, e_pad)
    msgs = _tc_edge_msgs(gsrc, gdst, ein, w, e_pad, e, blk_e)
    zrows = jnp.zeros(((n_pad // 4 + 16) // _NS, 128), F32)
    agg = _sc_scatter(msgs, dst3, zrows, n_pad, e_pad)
    out = _tc_final(nf, agg, w, n_pad, blk_n)
    return out[:n]


# speedup vs baseline: 2.4064x; 2.0381x over previous
import functools

import jax
import jax.numpy as jnp
from jax import lax
from jax.experimental import pallas as pl
from jax.experimental.pallas import tpu as pltpu
from jax.experimental.pallas import tpu_sc as plsc

F32 = jnp.float32


def _mm(a, w):
    return lax.dot_general(a, w, (((1,), (0,)), ((), ())),
                           preferred_element_type=F32)


def _rss(vh, nch):
    s2 = vh[:, :nch] ** 2 + vh[:, nch:2 * nch] ** 2 + vh[:, 2 * nch:] ** 2
    return jnp.sqrt(jnp.maximum(s2, 1e-8))


def _ln_s(s, g, b):
    mu = jnp.mean(s, axis=-1, keepdims=True)
    var = jnp.mean((s - mu) ** 2, axis=-1, keepdims=True)
    return (s - mu) / jnp.sqrt(var + 1e-5) * g + b


def _ln_v(v, nch):
    s2 = jnp.maximum(v[:, :nch] ** 2 + v[:, nch:2 * nch] ** 2
                     + v[:, 2 * nch:] ** 2, 1e-8)
    rms = jnp.sqrt(jnp.mean(s2, axis=-1, keepdims=True))
    return v / rms


def _node_math(x, w):
    xs, xv, nt = x[:, 0:6], x[:, 6:15], x[:, 15]
    vh = _mm(xv, w['np_wh'])
    vn = _rss(vh, 8)
    s = _mm(jnp.concatenate([xs, vn], 1), w['np_ws']) + w['np_bs']
    v = _mm(vh, w['np_wv'])
    s = _ln_s(s, w['npln_g'], w['npln_b'])
    v = _ln_v(v, 8)
    ids = lax.broadcasted_iota(jnp.int32, (x.shape[0], 20), 1).astype(F32)
    oh = (nt[:, None] == ids).astype(F32)
    s = jnp.concatenate([_mm(oh, w['ntype_emb']), s], 1)
    s = _ln_s(s, w['gnln_g'], w['gnln_b'])
    v = _ln_v(v, 8)
    vh = _mm(v, w['gn_wh'])
    vn = _rss(vh, 8)
    s = _mm(jnp.concatenate([s, vn], 1), w['gn_ws']) + w['gn_bs']
    v = _mm(vh, w['gn_wv'])
    return jnp.concatenate(
        [s, v, jnp.zeros((x.shape[0], 40), F32)], 1)


def _edge_math(gs, gd, ein, w):
    s_src, v_src = gs[:, 0:64], gs[:, 64:88]
    s_dst, v_dst = gd[:, 0:64], gd[:, 64:88]
    es_a, ev, et = ein[:, 0:32], ein[:, 32:35], ein[:, 35]
    vh = ev * w['ep_wh']
    vn = _rss(vh, 1)
    es = _mm(jnp.concatenate([es_a, vn], 1), w['ep_ws']) + w['ep_bs']
    ev1 = vh * w['ep_wv']
    es = _ln_s(es, w['epln_g'], w['epln_b'])
    ev1 = _ln_v(ev1, 1)
    ids = lax.broadcasted_iota(jnp.int32, (ein.shape[0], 4), 1).astype(F32)
    oh = (et[:, None] == ids).astype(F32)
    es = jnp.concatenate([_mm(oh, w['etype_emb']), es], 1)
    es = _ln_s(es, w['geln_g'], w['geln_b'])
    ev1 = _ln_v(ev1, 1)
    vh = ev1 * w['ge_wh']
    vn = _rss(vh, 1)
    es = _mm(jnp.concatenate([es, vn], 1), w['ge_ws']) + w['ge_bs']
    ev2 = vh * w['ge_wv']
    ms = jnp.concatenate([s_src, es, s_dst], 1)
    mv = jnp.concatenate([
        v_src[:, 0:8], ev2[:, 0:1], v_dst[:, 0:8],
        v_src[:, 8:16], ev2[:, 1:2], v_dst[:, 8:16],
        v_src[:, 16:24], ev2[:, 2:3], v_dst[:, 16:24]], 1)
    vh = _mm(mv, w['m0_wh'])
    vn = _rss(vh, 17)
    ms = _mm(jnp.concatenate([ms, vn], 1), w['m0_ws']) + w['m0_bs']
    v = _mm(vh, w['m0_wv'])
    vh = _mm(v, w['m1_wh'])
    vn = _rss(vh, 8)
    ms = _mm(jnp.concatenate([ms, vn], 1), w['m1_ws']) + w['m1_bs']
    v = _mm(vh, w['m1_wv'])
    vh = _mm(v, w['m2_wh'])
    vn = _rss(vh, 8)
    ms = _mm(jnp.concatenate([ms, vn], 1), w['m2_ws']) + w['m2_bs']
    v = _mm(vh, w['m2_wv'])
    B = gs.shape[0]
    return jnp.concatenate(
        [ms, v, jnp.ones((B, 1), F32), jnp.zeros((B, 39), F32)], 1)


def _final_math(nf, ag, w):
    s0, v0 = nf[:, 0:64], nf[:, 64:88]
    cnt = jnp.maximum(ag[:, 88:89], 1.0)
    s = s0 + ag[:, 0:64] / cnt
    v = v0 + ag[:, 64:88] / cnt
    s = _ln_s(s, w['n0_g'], w['n0_b'])
    v = _ln_v(v, 8)
    vh = _mm(v, w['f0_wh'])
    vn = _rss(vh, 16)
    fs = _mm(jnp.concatenate([s, vn], 1), w['f0_ws']) + w['f0_bs']
    fv = _mm(vh, w['f0_wv'])
    vh = _mm(fv, w['f1_wh'])
    vn = _rss(vh, 16)
    fs = _mm(jnp.concatenate([fs, vn], 1), w['f1_ws']) + w['f1_bs']
    fv = _mm(vh, w['f1_wv'])
    s = _ln_s(s + fs, w['n1_g'], w['n1_b'])
    v = _ln_v(v + fv, 8)
    s = _ln_s(s, w['fl_g'], w['fl_b'])
    v = _ln_v(v, 8)
    vh = _mm(v, w['o_wh'])
    vn = _rss(vh, 8)
    return _mm(jnp.concatenate([s, vn], 1), w['o_ws']) + w['o_bs']


def _prep_weights(p):
    I3 = jnp.eye(3, dtype=F32)

    def bd(W):
        return jnp.kron(I3, W)

    def row(x):
        return x.reshape(1, -1).astype(F32)

    c = p['convs'][0]
    return {
        'np_wh': bd(p['np_gvp']['Wh']), 'np_ws': p['np_gvp']['Ws'],
        'np_bs': row(p['np_gvp']['bs']), 'np_wv': bd(p['np_gvp']['Wv']),
        'npln_g': row(p['np_ln']['g']), 'npln_b': row(p['np_ln']['b']),
        'ntype_emb': p['ntype_emb'],
        'gnln_g': row(p['gn_ln']['g']), 'gnln_b': row(p['gn_ln']['b']),
        'gn_wh': bd(p['gn_gvp']['Wh']), 'gn_ws': p['gn_gvp']['Ws'],
        'gn_bs': row(p['gn_gvp']['bs']), 'gn_wv': bd(p['gn_gvp']['Wv']),
        'ep_wh': row(p['ep_gvp']['Wh']), 'ep_ws': p['ep_gvp']['Ws'],
        'ep_bs': row(p['ep_gvp']['bs']), 'ep_wv': row(p['ep_gvp']['Wv']),
        'epln_g': row(p['ep_ln']['g']), 'epln_b': row(p['ep_ln']['b']),
        'etype_emb': p['etype_emb'],
        'geln_g': row(p['ge_ln']['g']), 'geln_b': row(p['ge_ln']['b']),
        'ge_wh': row(p['ge_gvp']['Wh']), 'ge_ws': p['ge_gvp']['Ws'],
        'ge_bs': row(p['ge_gvp']['bs']), 'ge_wv': row(p['ge_gvp']['Wv']),
        'm0_wh': bd(c['msg0']['Wh']), 'm0_ws': c['msg0']['Ws'],
        'm0_bs': row(c['msg0']['bs']), 'm0_wv': bd(c['msg0']['Wv']),
        'm1_wh': bd(c['msg1']['Wh']), 'm1_ws': c['msg1']['Ws'],
        'm1_bs': row(c['msg1']['bs']), 'm1_wv': bd(c['msg1']['Wv']),
        'm2_wh': bd(c['msg2']['Wh']), 'm2_ws': c['msg2']['Ws'],
        'm2_bs': row(c['msg2']['bs']), 'm2_wv': bd(c['msg2']['Wv']),
        'n0_g': row(c['norm0']['g']), 'n0_b': row(c['norm0']['b']),
        'f0_wh': bd(c['ff0']['Wh']), 'f0_ws': c['ff0']['Ws'],
        'f0_bs': row(c['ff0']['bs']), 'f0_wv': bd(c['ff0']['Wv']),
        'f1_wh': bd(c['ff1']['Wh']), 'f1_ws': c['ff1']['Ws'],
        'f1_bs': row(c['ff1']['bs']), 'f1_wv': bd(c['ff1']['Wv']),
        'n1_g': row(c['norm1']['g']), 'n1_b': row(c['norm1']['b']),
        'fl_g': row(p['final_ln']['g']), 'fl_b': row(p['final_ln']['b']),
        'o_wh': bd(p['out_gvp']['Wh']), 'o_ws': p['out_gvp']['Ws'],
        'o_bs': row(p['out_gvp']['bs']),
    }


_A_KEYS = ['np_wh', 'np_ws', 'np_bs', 'np_wv', 'npln_g', 'npln_b',
           'ntype_emb', 'gnln_g', 'gnln_b', 'gn_wh', 'gn_ws', 'gn_bs',
           'gn_wv']
_C_KEYS = ['ep_wh', 'ep_ws', 'ep_bs', 'ep_wv', 'epln_g', 'epln_b',
           'etype_emb', 'geln_g', 'geln_b', 'ge_wh', 'ge_ws', 'ge_bs',
           'ge_wv',
           'm0_wh', 'm0_ws', 'm0_bs', 'm0_wv',
           'm1_wh', 'm1_ws', 'm1_bs', 'm1_wv',
           'm2_wh', 'm2_ws', 'm2_bs', 'm2_wv']
_E_KEYS = ['n0_g', 'n0_b', 'f0_wh', 'f0_ws', 'f0_bs', 'f0_wv',
           'f1_wh', 'f1_ws', 'f1_bs', 'f1_wv', 'n1_g', 'n1_b',
           'fl_g', 'fl_b', 'o_wh', 'o_ws', 'o_bs']


def _full_spec(x):
    return pl.BlockSpec(x.shape, lambda i: tuple(0 for _ in x.shape))


def _tc_node_prep(nodes_in, w, n_pad, blk):
    def body(xin_ref, *refs):
        wrefs, out_ref = refs[:-1], refs[-1]
        wd = {k: r[...] for k, r in zip(_A_KEYS, wrefs)}
        out_ref[...] = _node_math(xin_ref[...], wd)

    grid = (n_pad // blk,)
    return pl.pallas_call(
        body,
        grid=grid,
        in_specs=[pl.BlockSpec((blk, 16), lambda i: (i, 0))] +
                 [_full_spec(w[k]) for k in _A_KEYS],
        out_specs=pl.BlockSpec((blk, 128), lambda i: (i, 0)),
        out_shape=jax.ShapeDtypeStruct((n_pad, 128), F32),
    )(nodes_in, *[w[k] for k in _A_KEYS])


def _tc_edge_msgs(gsrc, gdst, ein, w, e_pad, e_real, blk):
    def body(gs_ref, gd_ref, ein_ref, *refs):
        wrefs, out_ref = refs[:-1], refs[-1]
        wd = {k: r[...] for k, r in zip(_C_KEYS, wrefs)}
        m = _edge_math(gs_ref[...], gd_ref[...], ein_ref[...], wd)
        i = pl.program_id(0)
        rows = i * blk + lax.broadcasted_iota(jnp.int32, (blk, 1), 0)
        out_ref[...] = m * (rows < e_real).astype(F32)

    grid = (e_pad // blk,)
    return pl.pallas_call(
        body,
        grid=grid,
        in_specs=[pl.BlockSpec((blk, 128), lambda i: (i, 0)),
                  pl.BlockSpec((blk, 128), lambda i: (i, 0)),
                  pl.BlockSpec((blk, 36), lambda i: (i, 0))] +
                 [_full_spec(w[k]) for k in _C_KEYS],
        out_specs=pl.BlockSpec((blk, 128), lambda i: (i, 0)),
        out_shape=jax.ShapeDtypeStruct((e_pad, 128), F32),
    )(gsrc, gdst, ein, *[w[k] for k in _C_KEYS])


def _tc_final(nf, agg, w, n_pad, blk):
    def body(nf_ref, ag_ref, *refs):
        wrefs, out_ref = refs[:-1], refs[-1]
        wd = {k: r[...] for k, r in zip(_E_KEYS, wrefs)}
        out_ref[...] = _final_math(nf_ref[...], ag_ref[...], wd)

    grid = (n_pad // blk,)
    return pl.pallas_call(
        body,
        grid=grid,
        in_specs=[pl.BlockSpec((blk, 128), lambda i: (i, 0)),
                  pl.BlockSpec((blk, 128), lambda i: (i, 0))] +
                 [_full_spec(w[k]) for k in _E_KEYS],
        out_specs=pl.BlockSpec((blk, 8), lambda i: (i, 0)),
        out_shape=jax.ShapeDtypeStruct((n_pad, 8), F32),
    )(nf, agg, *[w[k] for k in _E_KEYS])


_NC, _NS, _CH = 2, 16, 128


def _sc_gather(nf, src3d, dst3d, e_pad):
    nw = _NC * _NS
    n_chunks = e_pad // (nw * _CH)
    mesh = plsc.VectorSubcoreMesh(core_axis_name="c", subcore_axis_name="s")

    @functools.partial(
        pl.kernel, mesh=mesh,
        out_type=[jax.ShapeDtypeStruct((e_pad, 128), F32),
                  jax.ShapeDtypeStruct((e_pad, 128), F32)],
        scratch_types=[pltpu.VMEM((n_chunks, _CH), jnp.int32),
                       pltpu.VMEM((n_chunks, _CH), jnp.int32),
                       pltpu.VMEM((_CH, 128), F32),
                       pltpu.VMEM((_CH, 128), F32),
                       pltpu.SemaphoreType.DMA,
                       pltpu.SemaphoreType.DMA])
    def k(nf_hbm, src_hbm, dst_hbm, gsrc_hbm, gdst_hbm,
          src_v, dst_v, b0, b1, sem0, sem1):
        wid = lax.axis_index("s") * _NC + lax.axis_index("c")
        c0 = wid * n_chunks
        pltpu.sync_copy(src_hbm.at[wid], src_v)
        pltpu.sync_copy(dst_hbm.at[wid], dst_v)

        def body(j, carry):
            cp0 = pltpu.async_copy(nf_hbm.at[src_v.at[j]], b0, sem0)
            cp1 = pltpu.async_copy(nf_hbm.at[dst_v.at[j]], b1, sem1)
            cp0.wait()
            pltpu.sync_copy(b0, gsrc_hbm.at[pl.ds((c0 + j) * _CH, _CH)])
            cp1.wait()
            pltpu.sync_copy(b1, gdst_hbm.at[pl.ds((c0 + j) * _CH, _CH)])
            return carry

        lax.fori_loop(0, n_chunks, body, 0)

    return k(nf, src3d, dst3d)


def _sc_scatter(msgs, dst3, zrows, n_pad, e_pad):
    chunks_total = e_pad // _CH
    cpt = chunks_total // _NS
    nsub = n_pad // 4
    acc_rows = nsub + 16
    zrt = acc_rows // _NS
    wrt = nsub // _NS
    mesh = plsc.VectorSubcoreMesh(core_axis_name="c", subcore_axis_name="s")

    @functools.partial(
        pl.kernel, mesh=mesh,
        out_type=jax.ShapeDtypeStruct((n_pad, 128), F32),
        scratch_types=[pltpu.VMEM((1, 1, _CH), jnp.int32),
                       pltpu.VMEM((1, 1, _CH), jnp.int32),
                       pltpu.VMEM((_CH, 128), F32),
                       pltpu.VMEM_SHARED((acc_rows, 128), F32),
                       pltpu.SemaphoreType.DMA,
                       pltpu.SemaphoreType.DMA])
    def k(msgs_hbm, dst_hbm, z_hbm, out_hbm, idxb, ridx, mb0, acc, sm0, si0):
        c = lax.axis_index("c")
        s = lax.axis_index("s")

        for kk in range(2):
            r = 2 * c + kk
            base = r * nsub
            pltpu.sync_copy(z_hbm.at[pl.ds(0, zrt)],
                            acc.at[pl.ds(s * zrt, zrt)])
            plsc.subcore_barrier()

            def body(j, carry):
                row0 = (s * cpt + j) * _CH
                cp = pltpu.async_copy(msgs_hbm.at[pl.ds(row0, _CH)],
                                      mb0, sm0)
                ci = pltpu.async_copy(dst_hbm.at[pl.ds(s * cpt + j, 1)],
                                      idxb, si0)
                ci.wait()
                for l in range(_CH // 16):
                    iv = idxb[0, 0, pl.ds(16 * l, 16)]
                    loc = iv - base
                    ok = (loc >= 0) & (loc < nsub)
                    ridx[0, 0, pl.ds(16 * l, 16)] = jnp.where(
                        ok, loc, nsub)
                cp.wait()
                pltpu.sync_copy(mb0, acc.at[ridx.at[0, 0]], add=True)
                return carry

            lax.fori_loop(0, cpt, body, 0)
            plsc.subcore_barrier()
            pltpu.sync_copy(
                acc.at[pl.ds(s * wrt, wrt)],
                out_hbm.at[pl.ds(base + s * wrt, wrt)])
            plsc.subcore_barrier()

    return k(msgs, dst3, zrows)


def kernel(x_s, x_v, edge_index, ntypes, etypes, eattr_s, eattr_v, params):
    n = x_s.shape[0]
    e = edge_index.shape[1]
    blk_n, blk_e = 256, 512
    n_pad = -(-n // 256) * 256
    e_pad = -(-e // 4096) * 4096
    w = _prep_weights(params)

    xv_flat = jnp.transpose(x_v, (0, 2, 1)).reshape(n, 9)
    nodes_in = jnp.concatenate(
        [x_s, xv_flat, ntypes.astype(F32)[:, None]], axis=1)
    nodes_in = jnp.pad(nodes_in, ((0, n_pad - n), (0, 0)))

    ein = jnp.concatenate(
        [eattr_s, eattr_v.reshape(e, 3), etypes.astype(F32)[:, None]], axis=1)
    ein = jnp.pad(ein, ((0, e_pad - e), (0, 0)))

    src = jnp.pad(edge_index[0].astype(jnp.int32), (0, e_pad - e))
    dst = jnp.pad(edge_index[1].astype(jnp.int32), (0, e_pad - e))
    nw = _NC * _NS
    src3d = src.reshape(nw, e_pad // (nw * _CH), _CH)
    dst3d = dst.reshape(nw, e_pad // (nw * _CH), _CH)
    dst3 = dst.reshape(e_pad // _CH, 1, _CH)

    nf = _tc_node_prep(nodes_in, w, n_pad, blk_n)
    gsrc, gdst = _sc_gather(nf, src3d, dst3d, e_pad)
    msgs = _tc_edge_msgs(gsrc, gdst, ein, w, e_pad, e, blk_e)
    zrows = jnp.zeros(((n_pad // 4 + 16) // _NS, 128), F32)
    agg = _sc_scatter(msgs, dst3, zrows, n_pad, e_pad)
    out = _tc_final(nf, agg, w, n_pad, blk_n)
    return out[:n]
